# baseline (speedup 1.0000x reference)
import functools

import jax
import jax.numpy as jnp
from jax import lax
from jax.experimental import pallas as pl
from jax.experimental.pallas import tpu as pltpu
from jax.experimental.pallas import tpu_sc as plsc

N = 10000
E = 320000
D = 128
N_PAD = 10240
E_PAD = 327680
NW = 32
EPW = E_PAD // NW
CH = 128
NCH = EPW // CH
RPT = N_PAD // 16
EH = E_PAD // 2

_F32 = jnp.float32


def _ln(y, g, b):
    mu = jnp.mean(y, axis=-1, keepdims=True)
    yc = y - mu
    var = jnp.mean(yc * yc, axis=-1, keepdims=True)
    return yc / jnp.sqrt(var + 1e-5) * g + b


def _dot(a, b):
    return jnp.dot(a.astype(jnp.bfloat16), b.astype(jnp.bfloat16),
                   preferred_element_type=_F32)


def _mlp3_ln_body(x_ref, w0, b0, w1, b1, w2, b2, lg, lb, o_ref):
    h = jnp.maximum(_dot(x_ref[...], w0[...]) + b0[...], 0.0)
    h = jnp.maximum(_dot(h, w1[...]) + b1[...], 0.0)
    y = _dot(h, w2[...]) + b2[...]
    o_ref[...] = _ln(y, lg[...], lb[...])


def _mlp3_body(x_ref, w0, b0, w1, b1, w2, b2, o_ref):
    h = jnp.maximum(_dot(x_ref[...], w0[...]) + b0[...], 0.0)
    h = jnp.maximum(_dot(h, w1[...]) + b1[...], 0.0)
    o_ref[...] = _dot(h, w2[...]) + b2[...]


def _edge_mlp_body(gs_ref, gd_ref, e_ref, w1, b1, w2, b2, w3, b3, lg, lb,
                   o_ref):
    e = e_ref[...]
    x = jnp.concatenate([gs_ref[...], gd_ref[...], e], axis=-1)
    h = jnp.maximum(_dot(x, w1[...]) + b1[...], 0.0)
    h = jnp.maximum(_dot(h, w2[...]) + b2[...], 0.0)
    y = _dot(h, w3[...]) + b3[...]
    o_ref[...] = _ln(y, lg[...], lb[...]) + e


def _node_mlp_body(n_ref, a0_ref, a1_ref, a2_ref, a3_ref, w1, b1, w2, b2,
                   w3, b3, lg, lb, on_ref):
    n = n_ref[...]
    a = (a0_ref[...] + a1_ref[...]) + (a2_ref[...] + a3_ref[...])
    x = jnp.concatenate([n, a], axis=-1)
    h = jnp.maximum(_dot(x, w1[...]) + b1[...], 0.0)
    h = jnp.maximum(_dot(h, w2[...]) + b2[...], 0.0)
    y = _dot(h, w3[...]) + b3[...]
    on_ref[...] = _ln(y, lg[...], lb[...]) + n


def _full(shape):
    return pl.BlockSpec(shape, lambda i: (0,) * len(shape))


def _rows(bm, ncols):
    return pl.BlockSpec((bm, ncols), lambda i: (i, 0))


def _make_mlp3(nrows, bm, din, body):
    grid = (nrows // bm,)
    n_w = 8 if body is _mlp3_ln_body else 6
    wspecs = []
    dims = [din, D, D]
    for j in range(3):
        wspecs += [_full((dims[j], D)), _full((1, D))]
    wspecs += [_full((1, D))] * (n_w - 6)
    return pl.pallas_call(
        body,
        grid=grid,
        in_specs=[_rows(bm, din)] + wspecs,
        out_specs=_rows(bm, D),
        out_shape=jax.ShapeDtypeStruct((nrows, D), _F32),
    )


_enc_node = _make_mlp3(N_PAD, 2048, D, _mlp3_ln_body)
_enc_edge = _make_mlp3(E_PAD, 4096, 16, _mlp3_ln_body)
_decoder = _make_mlp3(N_PAD, 2048, D, _mlp3_body)

_BE = 1024
_edge_mlp = pl.pallas_call(
    _edge_mlp_body,
    grid=(EH // _BE,),
    in_specs=[_rows(_BE, D), _rows(_BE, D), _rows(_BE, D),
              _full((3 * D, D)), _full((1, D)),
              _full((D, D)), _full((1, D)),
              _full((D, D)), _full((1, D)),
              _full((1, D)), _full((1, D))],
    out_specs=_rows(_BE, D),
    out_shape=jax.ShapeDtypeStruct((EH, D), _F32),
)

_BN = 2048
_node_mlp = pl.pallas_call(
    _node_mlp_body,
    grid=(N_PAD // _BN,),
    in_specs=[_rows(_BN, D), _rows(_BN, D), _rows(_BN, D),
              _rows(_BN, D), _rows(_BN, D),
              _full((2 * D, D)), _full((1, D)),
              _full((D, D)), _full((1, D)),
              _full((D, D)), _full((1, D)),
              _full((1, D)), _full((1, D))],
    out_specs=_rows(_BN, D),
    out_shape=jax.ShapeDtypeStruct((N_PAD, D), _F32),
)


_sc_mesh = plsc.VectorSubcoreMesh(core_axis_name="c", subcore_axis_name="s")


def _make_gather(ne):
    nch = ne // NW // CH
    epw = ne // NW

    @functools.partial(
        pl.kernel,
        out_type=(jax.ShapeDtypeStruct((ne, D), _F32),
                  jax.ShapeDtypeStruct((ne, D), _F32)),
        mesh=_sc_mesh,
        scratch_types=[pltpu.VMEM((nch, CH), jnp.int32),
                       pltpu.VMEM((nch, CH), jnp.int32),
                       pltpu.VMEM((2, CH, D), _F32),
                       pltpu.VMEM((2, CH, D), _F32),
                       [pltpu.SemaphoreType.DMA] * 2,
                       [pltpu.SemaphoreType.DMA] * 2,
                       [pltpu.SemaphoreType.DMA] * 2,
                       [pltpu.SemaphoreType.DMA] * 2],
    )
    def gather_k(ps_hbm, pd_hbm, src_hbm, dst_hbm, gs_hbm, gd_hbm,
                 srcv, dstv, rs, rd, gsem_s, gsem_d, ssem_s, ssem_d):
        wid = lax.axis_index("s") * 2 + lax.axis_index("c")
        base = wid * epw
        pltpu.sync_copy(src_hbm.at[pl.ds(wid * nch, nch)], srcv)
        pltpu.sync_copy(dst_hbm.at[pl.ds(wid * nch, nch)], dstv)

        def issue_gather(i, b):
            pltpu.async_copy(ps_hbm.at[srcv.at[i]], rs.at[b], gsem_s[b])
            pltpu.async_copy(pd_hbm.at[dstv.at[i]], rd.at[b], gsem_d[b])

        def wait_gather(b):
            pltpu.make_async_copy(ps_hbm.at[srcv.at[0]], rs.at[b],
                                  gsem_s[b]).wait()
            pltpu.make_async_copy(pd_hbm.at[dstv.at[0]], rd.at[b],
                                  gsem_d[b]).wait()

        def issue_store(i, b):
            off = base + i * CH
            pltpu.async_copy(rs.at[b], gs_hbm.at[pl.ds(off, CH)], ssem_s[b])
            pltpu.async_copy(rd.at[b], gd_hbm.at[pl.ds(off, CH)], ssem_d[b])

        def wait_store(b):
            pltpu.make_async_copy(rs.at[b], gs_hbm.at[pl.ds(0, CH)],
                                  ssem_s[b]).wait()
            pltpu.make_async_copy(rd.at[b], gd_hbm.at[pl.ds(0, CH)],
                                  ssem_d[b]).wait()

        issue_gather(0, 0)

        @pl.loop(0, nch // 2)
        def _pair(j):
            i0 = 2 * j

            @pl.when(j > 0)
            def _():
                wait_store(1)

            issue_gather(i0 + 1, 1)
            wait_gather(0)
            issue_store(i0, 0)

            @pl.when(j < nch // 2 - 1)
            def _():
                wait_store(0)
                issue_gather(i0 + 2, 0)

            wait_gather(1)
            issue_store(i0 + 1, 1)

        wait_store(0)
        wait_store(1)

    return gather_k


def _make_scatter(ne):
    nch = ne // NW // CH
    epw = ne // NW

    @functools.partial(
        pl.kernel,
        out_type=jax.ShapeDtypeStruct((2, N_PAD, D), _F32),
        mesh=_sc_mesh,
        scratch_types=[pltpu.VMEM((nch, CH), jnp.int32),
                       pltpu.VMEM((2, CH, D), _F32),
                       pltpu.VMEM_SHARED((N_PAD, D), _F32),
                       [pltpu.SemaphoreType.DMA] * 2,
                       [pltpu.SemaphoreType.DMA] * 2],
    )
    def scatter_k(enew_hbm, dst_hbm, zeros_hbm, out_hbm, dstv, rows, acc,
                  lsem, asem):
        c = lax.axis_index("c")
        s = lax.axis_index("s")
        wid = s * 2 + c
        base = wid * epw
        pltpu.sync_copy(dst_hbm.at[pl.ds(wid * nch, nch)], dstv)
        pltpu.sync_copy(zeros_hbm.at[pl.ds(s * RPT, RPT)],
                        acc.at[pl.ds(s * RPT, RPT)])
        plsc.subcore_barrier()

        def issue_load(i, b):
            pltpu.async_copy(enew_hbm.at[pl.ds(base + i * CH, CH)],
                             rows.at[b], lsem[b])

        def wait_load(b):
            pltpu.make_async_copy(enew_hbm.at[pl.ds(0, CH)], rows.at[b],
                                  lsem[b]).wait()

        def fire_add(i, b):
            pltpu.async_copy(rows.at[b], acc.at[dstv.at[i]], asem[b],
                             add=True)

        def wait_add(b):
            pltpu.make_async_copy(rows.at[b], acc.at[dstv.at[0]],
                                  asem[b]).wait()

        issue_load(0, 0)
        issue_load(1, 1)

        @pl.loop(0, nch // 2)
        def _pair(j):
            i0 = 2 * j
            wait_load(0)
            fire_add(i0, 0)
            wait_load(1)
            fire_add(i0 + 1, 1)

            @pl.when(j < nch // 2 - 1)
            def _():
                wait_add(0)
                issue_load(i0 + 2, 0)
                wait_add(1)
                issue_load(i0 + 3, 1)

        wait_add(0)
        wait_add(1)
        plsc.subcore_barrier()
        pltpu.sync_copy(acc.at[pl.ds(s * RPT, RPT)],
                        out_hbm.at[c].at[pl.ds(s * RPT, RPT)])

    return scatter_k


_gather_h = _make_gather(EH)
_scatter_h = _make_scatter(EH)


def _mlp_args(p):
    out = []
    for j in range(3):
        out.append(p['W'][j])
        out.append(p['b'][j].reshape(1, D))
    if 'ln_g' in p:
        out.append(p['ln_g'].reshape(1, D))
        out.append(p['ln_b'].reshape(1, D))
    return out


def kernel(params, x, edge_attr, edge_index):
    xp = jnp.zeros((N_PAD, D), _F32).at[:N].set(x)
    ea = jnp.zeros((E_PAD, 16), _F32).at[:E].set(edge_attr)
    src = jnp.zeros((E_PAD,), jnp.int32).at[:E].set(
        edge_index[0]).reshape(E_PAD // CH, CH)
    dst = jnp.full((E_PAD,), N_PAD - 1, jnp.int32).at[:E].set(
        edge_index[1]).reshape(E_PAD // CH, CH)
    hc = EH // CH
    src_a, src_b = src[:hc], src[hc:]
    dst_a, dst_b = dst[:hc], dst[hc:]
    zeros_n = jnp.zeros((N_PAD, D), _F32)

    node = _enc_node(xp, *_mlp_args(params['node_enc']))
    edge = _enc_edge(ea, *_mlp_args(params['edge_enc']))
    edge_a, edge_b = edge[:EH], edge[EH:]

    for i in range(15):
        ep = params['edge_proc'][i]
        np_ = params['node_proc'][i]
        emlp_w = (ep['W'][0], ep['b'][0].reshape(1, D),
                  ep['W'][1], ep['b'][1].reshape(1, D),
                  ep['W'][2], ep['b'][2].reshape(1, D),
                  ep['ln_g'].reshape(1, D), ep['ln_b'].reshape(1, D))
        gs_a, gd_a = _gather_h(node, node, src_a, dst_a)
        gs_b, gd_b = _gather_h(node, node, src_b, dst_b)
        enew_a = _edge_mlp(gs_a, gd_a, edge_a, *emlp_w)
        enew_b = _edge_mlp(gs_b, gd_b, edge_b, *emlp_w)
        parts_a = _scatter_h(enew_a, dst_a, zeros_n)
        parts_b = _scatter_h(enew_b, dst_b, zeros_n)
        node = _node_mlp(
            node, parts_a[0], parts_a[1], parts_b[0], parts_b[1],
            np_['W'][0], np_['b'][0].reshape(1, D),
            np_['W'][1], np_['b'][1].reshape(1, D),
            np_['W'][2], np_['b'][2].reshape(1, D),
            np_['ln_g'].reshape(1, D), np_['ln_b'].reshape(1, D))
        edge_a, edge_b = enew_a, enew_b

    dp = params['node_dec']
    w2 = jnp.zeros((D, D), _F32).at[:, :3].set(dp['W'][2])
    b2 = jnp.zeros((1, D), _F32).at[0, :3].set(dp['b'][2])
    out = _decoder(
        node,
        dp['W'][0], dp['b'][0].reshape(1, D),
        dp['W'][1], dp['b'][1].reshape(1, D),
        w2, b2)
    return out[:N, :3]


# speedup vs baseline: 1.0363x; 1.0363x over previous
import functools

import jax
import jax.numpy as jnp
from jax import lax
from jax.experimental import pallas as pl
from jax.experimental.pallas import tpu as pltpu
from jax.experimental.pallas import tpu_sc as plsc

N = 10000
E = 320000
D = 128
N_PAD = 10240
E_PAD = 327680
NW = 32
EPW = E_PAD // NW
CH = 128
NCH = EPW // CH
RPT = N_PAD // 16
EH = E_PAD // 2

_F32 = jnp.float32


def _ln(y, g, b):
    mu = jnp.mean(y, axis=-1, keepdims=True)
    yc = y - mu
    var = jnp.mean(yc * yc, axis=-1, keepdims=True)
    return yc * lax.rsqrt(var + 1e-5) * g + b


def _dot(a, b):
    return jnp.dot(a.astype(jnp.bfloat16), b.astype(jnp.bfloat16),
                   preferred_element_type=_F32)


def _mlp3_ln_body(x_ref, w0, b0, w1, b1, w2, b2, lg, lb, o_ref):
    h = jnp.maximum(_dot(x_ref[...], w0[...]) + b0[...], 0.0)
    h = jnp.maximum(_dot(h, w1[...]) + b1[...], 0.0)
    y = _dot(h, w2[...]) + b2[...]
    o_ref[...] = _ln(y, lg[...], lb[...])


def _mlp3_body(x_ref, w0, b0, w1, b1, w2, b2, o_ref):
    h = jnp.maximum(_dot(x_ref[...], w0[...]) + b0[...], 0.0)
    h = jnp.maximum(_dot(h, w1[...]) + b1[...], 0.0)
    o_ref[...] = _dot(h, w2[...]) + b2[...]


def _edge_mlp_body(gs_ref, gd_ref, e_ref, we, b1, w2, b2, w3, b3, lg, lb,
                   o_ref):
    e = e_ref[...]
    h = gs_ref[...] + gd_ref[...] + _dot(e, we[...]) + b1[...]
    h = jnp.maximum(h, 0.0)
    h = jnp.maximum(_dot(h, w2[...]) + b2[...], 0.0)
    y = _dot(h, w3[...]) + b3[...]
    o_ref[...] = _ln(y, lg[...], lb[...]) + e


def _node_mlp_body(n_ref, a0_ref, a1_ref, a2_ref, a3_ref, wn, wa, b1, w2, b2,
                   w3, b3, lg, lb, wsn, wdn, on_ref, ops_ref, opd_ref):
    n = n_ref[...]
    a = (a0_ref[...] + a1_ref[...]) + (a2_ref[...] + a3_ref[...])
    h = jnp.maximum(_dot(n, wn[...]) + _dot(a, wa[...]) + b1[...], 0.0)
    h = jnp.maximum(_dot(h, w2[...]) + b2[...], 0.0)
    y = _dot(h, w3[...]) + b3[...]
    nn = _ln(y, lg[...], lb[...]) + n
    on_ref[...] = nn
    ops_ref[...] = _dot(nn, wsn[...])
    opd_ref[...] = _dot(nn, wdn[...])


def _project_body(n_ref, wsn, wdn, ops_ref, opd_ref):
    n = n_ref[...]
    ops_ref[...] = _dot(n, wsn[...])
    opd_ref[...] = _dot(n, wdn[...])


def _full(shape):
    return pl.BlockSpec(shape, lambda i: (0,) * len(shape))


def _rows(bm, ncols):
    return pl.BlockSpec((bm, ncols), lambda i: (i, 0))


def _make_mlp3(nrows, bm, din, body):
    grid = (nrows // bm,)
    n_w = 8 if body is _mlp3_ln_body else 6
    wspecs = []
    dims = [din, D, D]
    for j in range(3):
        wspecs += [_full((dims[j], D)), _full((1, D))]
    wspecs += [_full((1, D))] * (n_w - 6)
    return pl.pallas_call(
        body,
        grid=grid,
        in_specs=[_rows(bm, din)] + wspecs,
        out_specs=_rows(bm, D),
        out_shape=jax.ShapeDtypeStruct((nrows, D), _F32),
    )


_enc_node = _make_mlp3(N_PAD, 2048, D, _mlp3_ln_body)
_enc_edge = _make_mlp3(E_PAD, 4096, 16, _mlp3_ln_body)
_decoder = _make_mlp3(N_PAD, 2048, D, _mlp3_body)

_BE = 1024
_edge_mlp = pl.pallas_call(
    _edge_mlp_body,
    grid=(EH // _BE,),
    in_specs=[_rows(_BE, D), _rows(_BE, D), _rows(_BE, D),
              _full((D, D)), _full((1, D)),
              _full((D, D)), _full((1, D)),
              _full((D, D)), _full((1, D)),
              _full((1, D)), _full((1, D))],
    out_specs=_rows(_BE, D),
    out_shape=jax.ShapeDtypeStruct((EH, D), _F32),
)

_BN = 2048
_node_mlp = pl.pallas_call(
    _node_mlp_body,
    grid=(N_PAD // _BN,),
    in_specs=[_rows(_BN, D), _rows(_BN, D), _rows(_BN, D),
              _rows(_BN, D), _rows(_BN, D),
              _full((D, D)), _full((D, D)), _full((1, D)),
              _full((D, D)), _full((1, D)),
              _full((D, D)), _full((1, D)),
              _full((1, D)), _full((1, D)),
              _full((D, D)), _full((D, D))],
    out_specs=[_rows(_BN, D)] * 3,
    out_shape=[jax.ShapeDtypeStruct((N_PAD, D), _F32)] * 3,
)

_project = pl.pallas_call(
    _project_body,
    grid=(N_PAD // _BN,),
    in_specs=[_rows(_BN, D), _full((D, D)), _full((D, D))],
    out_specs=[_rows(_BN, D)] * 2,
    out_shape=[jax.ShapeDtypeStruct((N_PAD, D), _F32)] * 2,
)


_sc_mesh = plsc.VectorSubcoreMesh(core_axis_name="c", subcore_axis_name="s")


def _make_gather(ne):
    nch = ne // NW // CH
    epw = ne // NW

    @functools.partial(
        pl.kernel,
        out_type=(jax.ShapeDtypeStruct((ne, D), _F32),
                  jax.ShapeDtypeStruct((ne, D), _F32)),
        mesh=_sc_mesh,
        scratch_types=[pltpu.VMEM((nch, CH), jnp.int32),
                       pltpu.VMEM((nch, CH), jnp.int32),
                       pltpu.VMEM((2, CH, D), _F32),
                       pltpu.VMEM((2, CH, D), _F32),
                       [pltpu.SemaphoreType.DMA] * 2,
                       [pltpu.SemaphoreType.DMA] * 2,
                       [pltpu.SemaphoreType.DMA] * 2,
                       [pltpu.SemaphoreType.DMA] * 2],
    )
    def gather_k(ps_hbm, pd_hbm, src_hbm, dst_hbm, gs_hbm, gd_hbm,
                 srcv, dstv, rs, rd, gsem_s, gsem_d, ssem_s, ssem_d):
        wid = lax.axis_index("s") * 2 + lax.axis_index("c")
        base = wid * epw
        pltpu.sync_copy(src_hbm.at[pl.ds(wid * nch, nch)], srcv)
        pltpu.sync_copy(dst_hbm.at[pl.ds(wid * nch, nch)], dstv)

        def issue_gather(i, b):
            pltpu.async_copy(ps_hbm.at[srcv.at[i]], rs.at[b], gsem_s[b])
            pltpu.async_copy(pd_hbm.at[dstv.at[i]], rd.at[b], gsem_d[b])

        def wait_gather(b):
            pltpu.make_async_copy(ps_hbm.at[srcv.at[0]], rs.at[b],
                                  gsem_s[b]).wait()
            pltpu.make_async_copy(pd_hbm.at[dstv.at[0]], rd.at[b],
                                  gsem_d[b]).wait()

        def issue_store(i, b):
            off = base + i * CH
            pltpu.async_copy(rs.at[b], gs_hbm.at[pl.ds(off, CH)], ssem_s[b])
            pltpu.async_copy(rd.at[b], gd_hbm.at[pl.ds(off, CH)], ssem_d[b])

        def wait_store(b):
            pltpu.make_async_copy(rs.at[b], gs_hbm.at[pl.ds(0, CH)],
                                  ssem_s[b]).wait()
            pltpu.make_async_copy(rd.at[b], gd_hbm.at[pl.ds(0, CH)],
                                  ssem_d[b]).wait()

        issue_gather(0, 0)

        @pl.loop(0, nch // 2)
        def _pair(j):
            i0 = 2 * j

            @pl.when(j > 0)
            def _():
                wait_store(1)

            issue_gather(i0 + 1, 1)
            wait_gather(0)
            issue_store(i0, 0)

            @pl.when(j < nch // 2 - 1)
            def _():
                wait_store(0)
                issue_gather(i0 + 2, 0)

            wait_gather(1)
            issue_store(i0 + 1, 1)

        wait_store(0)
        wait_store(1)

    return gather_k


def _make_scatter(ne):
    nch = ne // NW // CH
    epw = ne // NW

    @functools.partial(
        pl.kernel,
        out_type=jax.ShapeDtypeStruct((2, N_PAD, D), _F32),
        mesh=_sc_mesh,
        scratch_types=[pltpu.VMEM((nch, CH), jnp.int32),
                       pltpu.VMEM((2, CH, D), _F32),
                       pltpu.VMEM_SHARED((N_PAD, D), _F32),
                       [pltpu.SemaphoreType.DMA] * 2,
                       [pltpu.SemaphoreType.DMA] * 2],
    )
    def scatter_k(enew_hbm, dst_hbm, zeros_hbm, out_hbm, dstv, rows, acc,
                  lsem, asem):
        c = lax.axis_index("c")
        s = lax.axis_index("s")
        wid = s * 2 + c
        base = wid * epw
        pltpu.sync_copy(dst_hbm.at[pl.ds(wid * nch, nch)], dstv)
        pltpu.sync_copy(zeros_hbm.at[pl.ds(s * RPT, RPT)],
                        acc.at[pl.ds(s * RPT, RPT)])
        plsc.subcore_barrier()

        def issue_load(i, b):
            pltpu.async_copy(enew_hbm.at[pl.ds(base + i * CH, CH)],
                             rows.at[b], lsem[b])

        def wait_load(b):
            pltpu.make_async_copy(enew_hbm.at[pl.ds(0, CH)], rows.at[b],
                                  lsem[b]).wait()

        def fire_add(i, b):
            pltpu.async_copy(rows.at[b], acc.at[dstv.at[i]], asem[b],
                             add=True)

        def wait_add(b):
            pltpu.make_async_copy(rows.at[b], acc.at[dstv.at[0]],
                                  asem[b]).wait()

        issue_load(0, 0)
        issue_load(1, 1)

        @pl.loop(0, nch // 2)
        def _pair(j):
            i0 = 2 * j
            wait_load(0)
            fire_add(i0, 0)
            wait_load(1)
            fire_add(i0 + 1, 1)

            @pl.when(j < nch // 2 - 1)
            def _():
                wait_add(0)
                issue_load(i0 + 2, 0)
                wait_add(1)
                issue_load(i0 + 3, 1)

        wait_add(0)
        wait_add(1)
        plsc.subcore_barrier()
        pltpu.sync_copy(acc.at[pl.ds(s * RPT, RPT)],
                        out_hbm.at[c].at[pl.ds(s * RPT, RPT)])

    return scatter_k


_gather_h = _make_gather(EH)
_scatter_h = _make_scatter(EH)


def _mlp_args(p):
    out = []
    for j in range(3):
        out.append(p['W'][j])
        out.append(p['b'][j].reshape(1, D))
    if 'ln_g' in p:
        out.append(p['ln_g'].reshape(1, D))
        out.append(p['ln_b'].reshape(1, D))
    return out


def kernel(params, x, edge_attr, edge_index):
    xp = jnp.zeros((N_PAD, D), _F32).at[:N].set(x)
    ea = jnp.zeros((E_PAD, 16), _F32).at[:E].set(edge_attr)
    src = jnp.zeros((E_PAD,), jnp.int32).at[:E].set(
        edge_index[0]).reshape(E_PAD // CH, CH)
    dst = jnp.full((E_PAD,), N_PAD - 1, jnp.int32).at[:E].set(
        edge_index[1]).reshape(E_PAD // CH, CH)
    hc = EH // CH
    src_a, src_b = src[:hc], src[hc:]
    dst_a, dst_b = dst[:hc], dst[hc:]
    zeros_n = jnp.zeros((N_PAD, D), _F32)

    node = _enc_node(xp, *_mlp_args(params['node_enc']))
    edge = _enc_edge(ea, *_mlp_args(params['edge_enc']))
    edge_a, edge_b = edge[:EH], edge[EH:]

    ew = []
    for i in range(15):
        p = params['edge_proc'][i]
        W0 = p['W'][0]
        ew.append((W0[:D], W0[D:2 * D], W0[2 * D:]))
    nw = []
    for i in range(15):
        p = params['node_proc'][i]
        W0 = p['W'][0]
        nw.append((W0[:D], W0[D:]))

    ps, pd = _project(node, ew[0][0], ew[0][1])
    for i in range(15):
        ep = params['edge_proc'][i]
        np_ = params['node_proc'][i]
        emlp_w = (ew[i][2], ep['b'][0].reshape(1, D),
                  ep['W'][1], ep['b'][1].reshape(1, D),
                  ep['W'][2], ep['b'][2].reshape(1, D),
                  ep['ln_g'].reshape(1, D), ep['ln_b'].reshape(1, D))
        gs_a, gd_a = _gather_h(ps, pd, src_a, dst_a)
        gs_b, gd_b = _gather_h(ps, pd, src_b, dst_b)
        enew_a = _edge_mlp(gs_a, gd_a, edge_a, *emlp_w)
        enew_b = _edge_mlp(gs_b, gd_b, edge_b, *emlp_w)
        parts_a = _scatter_h(enew_a, dst_a, zeros_n)
        parts_b = _scatter_h(enew_b, dst_b, zeros_n)
        j = min(i + 1, 14)
        node, ps, pd = _node_mlp(
            node, parts_a[0], parts_a[1], parts_b[0], parts_b[1],
            nw[i][0], nw[i][1], np_['b'][0].reshape(1, D),
            np_['W'][1], np_['b'][1].reshape(1, D),
            np_['W'][2], np_['b'][2].reshape(1, D),
            np_['ln_g'].reshape(1, D), np_['ln_b'].reshape(1, D),
            ew[j][0], ew[j][1])
        edge_a, edge_b = enew_a, enew_b

    dp = params['node_dec']
    w2 = jnp.zeros((D, D), _F32).at[:, :3].set(dp['W'][2])
    b2 = jnp.zeros((1, D), _F32).at[0, :3].set(dp['b'][2])
    out = _decoder(
        node,
        dp['W'][0], dp['b'][0].reshape(1, D),
        dp['W'][1], dp['b'][1].reshape(1, D),
        w2, b2)
    return out[:N, :3]
